# baseline (speedup 1.0000x reference)
import functools

import jax
import jax.numpy as jnp
from jax import lax
from jax.experimental import pallas as pl
from jax.experimental.pallas import tpu as pltpu
from jax.experimental.pallas import tpu_sc as plsc

B, S = 16384, 50
D = 32
V = 1000000
NC, NS = 2, 16
NW = NC * NS
NB_PER_W = B // NW
BCH = 128
NJ = NB_PER_W // BCH
NIT = S * NJ
NSLOT = 8
NGRP = NIT // NSLOT
PRO = NSLOT - 1
TPW = BCH + 1


def _gather_kernel(idx_hbm, table_hbm, out_hbm, idx_blk, idx_cols,
                   rows_v, tp_v, sem_i, sem_g, sem_o):
    wid = lax.axis_index("s") * NC + lax.axis_index("c")
    b_base = wid * NB_PER_W

    lane = jnp.arange(16, dtype=jnp.int32)
    r_lo = lane
    r_hi = lane + 16
    zero16 = jnp.zeros((16,), jnp.int32)
    flat50 = [(lane + k * 16) * S for k in range(NB_PER_W // 16)]

    pltpu.async_copy(
        idx_hbm.at[pl.ds(b_base * S, NB_PER_W * S)], idx_blk, sem_i
    ).wait()

    def tr_idx(s, carry):
        for k in range(NB_PER_W // 16):
            v = plsc.load_gather(idx_blk, [flat50[k] + s])
            idx_cols[s, pl.ds(k * 16, 16)] = v
        return carry

    lax.fori_loop(0, S, tr_idx, 0)

    def fire_gather(i, slot):
        s = i // NJ
        j = i - s * NJ
        pltpu.async_copy(
            table_hbm.at[idx_cols.at[s, pl.ds(j * BCH, BCH)]],
            rows_v.at[pl.ds(slot * BCH, BCH)],
            sem_g.at[slot],
        )

    def wait_gather(slot):
        pltpu.make_async_copy(
            table_hbm.at[pl.ds(0, BCH)],
            rows_v.at[pl.ds(slot * BCH, BCH)],
            sem_g.at[slot],
        ).wait()

    def fire_write(i, slot):
        s = i // NJ
        j = i - s * NJ
        pltpu.async_copy(
            tp_v.at[pl.ds(slot * D, D), pl.ds(0, BCH)],
            out_hbm.at[s, :, pl.ds(b_base + j * BCH, BCH)],
            sem_o.at[slot],
        )

    def wait_write(slot):
        pltpu.make_async_copy(
            tp_v.at[pl.ds(slot * D, D), pl.ds(0, BCH)],
            out_hbm.at[0, :, pl.ds(0, BCH)],
            sem_o.at[slot],
        ).wait()

    def transpose_block(slot):
        rows = rows_v.at[pl.ds(slot * BCH, BCH)]
        tp = tp_v.at[pl.ds(slot * D, D)]

        def h_body(h, carry):
            base = lax.mul(h, 16)
            for lp in range(16):
                l = base + lp
                col = zero16 + l
                v0 = rows[l, pl.ds(0, 16)]
                v1 = rows[l, pl.ds(16, 16)]
                plsc.store_scatter(tp, [r_lo, col], v0)
                plsc.store_scatter(tp, [r_hi, col], v1)
            return carry

        lax.fori_loop(0, BCH // 16, h_body, 0)

    for i in range(PRO):
        fire_gather(i, i)

    def body(g, carry):
        for p in range(NSLOT):
            i = g * NSLOT + p

            @pl.when(g >= 1)
            def _():
                wait_write(p)

            wait_gather(p)
            transpose_block(p)
            fire_write(i, p)
            ip = i + PRO

            @pl.when(ip < NIT)
            def _():
                fire_gather(ip, (p + PRO) % NSLOT)

        return carry

    lax.fori_loop(0, NGRP, body, 0)

    for slot in range(NSLOT):
        wait_write(slot)


@jax.jit
def _run(idx1d, table):
    mesh = plsc.VectorSubcoreMesh(core_axis_name="c", subcore_axis_name="s")
    f = functools.partial(
        pl.kernel,
        mesh=mesh,
        out_type=jax.ShapeDtypeStruct((S, D, B), jnp.float32),
        scratch_types=[
            pltpu.VMEM((NB_PER_W * S,), jnp.int32),
            pltpu.VMEM((S, NB_PER_W), jnp.int32),
            pltpu.VMEM((NSLOT * BCH, D), jnp.float32),
            pltpu.VMEM((NSLOT * D, TPW), jnp.float32),
            pltpu.SemaphoreType.DMA,
            pltpu.SemaphoreType.DMA((NSLOT,)),
            pltpu.SemaphoreType.DMA((NSLOT,)),
        ],
        compiler_params=pltpu.CompilerParams(
            use_tc_tiling_on_sc=False, needs_layout_passes=False
        ),
    )(_gather_kernel)
    return f(idx1d, table)


def kernel(indices, embeddings):
    idx1d = indices.astype(jnp.int32).reshape(-1)
    w3 = _run(idx1d, embeddings)
    return jnp.transpose(w3, (2, 0, 1))


# speedup vs baseline: 1.0105x; 1.0105x over previous
import functools

import jax
import jax.numpy as jnp
from jax import lax
from jax.experimental import pallas as pl
from jax.experimental.pallas import tpu as pltpu
from jax.experimental.pallas import tpu_sc as plsc

B, S = 16384, 50
D = 32
V = 1000000
NC, NS = 2, 16
NW = NC * NS
NB_PER_W = B // NW
BCH = 128
NJ = NB_PER_W // BCH
NIT = S * NJ
NSLOT = 8
NGRP = NIT // NSLOT
PRO = NSLOT - 1
TPW = BCH + 1


def _gather_kernel(idx_hbm, table_hbm, out_hbm, idx_cols,
                   rows_v, tp_v, sem_i, sem_g, sem_o):
    wid = lax.axis_index("s") * NC + lax.axis_index("c")
    b_base = wid * NB_PER_W

    lane = jnp.arange(16, dtype=jnp.int32)
    r_lo = lane
    r_hi = lane + 16
    zero16 = jnp.zeros((16,), jnp.int32)

    idx_copies = []
    for s in range(S):
        idx_copies.append(
            pltpu.async_copy(
                idx_hbm.at[pl.ds(s * B + b_base, NB_PER_W)],
                idx_cols.at[s],
                sem_i,
            )
        )
    for cp in idx_copies:
        cp.wait()

    def fire_gather(i, slot):
        s = i // NJ
        j = i - s * NJ
        pltpu.async_copy(
            table_hbm.at[idx_cols.at[s, pl.ds(j * BCH, BCH)]],
            rows_v.at[pl.ds(slot * BCH, BCH)],
            sem_g.at[slot],
        )

    def wait_gather(slot):
        pltpu.make_async_copy(
            table_hbm.at[pl.ds(0, BCH)],
            rows_v.at[pl.ds(slot * BCH, BCH)],
            sem_g.at[slot],
        ).wait()

    def fire_write(i, slot):
        s = i // NJ
        j = i - s * NJ
        pltpu.async_copy(
            tp_v.at[pl.ds(slot * D, D), pl.ds(0, BCH)],
            out_hbm.at[s, :, pl.ds(b_base + j * BCH, BCH)],
            sem_o.at[slot],
        )

    def wait_write(slot):
        pltpu.make_async_copy(
            tp_v.at[pl.ds(slot * D, D), pl.ds(0, BCH)],
            out_hbm.at[0, :, pl.ds(0, BCH)],
            sem_o.at[slot],
        ).wait()

    def transpose_block(slot):
        rows = rows_v.at[pl.ds(slot * BCH, BCH)]
        tp = tp_v.at[pl.ds(slot * D, D)]

        def h_body(h, carry):
            base = lax.mul(h, 16)
            for lp in range(16):
                l = base + lp
                col = zero16 + l
                v0 = rows[l, pl.ds(0, 16)]
                v1 = rows[l, pl.ds(16, 16)]
                plsc.store_scatter(tp, [r_lo, col], v0)
                plsc.store_scatter(tp, [r_hi, col], v1)
            return carry

        lax.fori_loop(0, BCH // 16, h_body, 0)

    for i in range(PRO):
        fire_gather(i, i)

    def body(g, carry):
        for p in range(NSLOT):
            i = g * NSLOT + p

            @pl.when(g >= 1)
            def _():
                wait_write(p)

            wait_gather(p)
            transpose_block(p)
            fire_write(i, p)
            ip = i + PRO

            @pl.when(ip < NIT)
            def _():
                fire_gather(ip, (p + PRO) % NSLOT)

        return carry

    lax.fori_loop(0, NGRP, body, 0)

    for slot in range(NSLOT):
        wait_write(slot)


@jax.jit
def _run(idx1d, table):
    mesh = plsc.VectorSubcoreMesh(core_axis_name="c", subcore_axis_name="s")
    f = functools.partial(
        pl.kernel,
        mesh=mesh,
        out_type=jax.ShapeDtypeStruct((S, D, B), jnp.float32),
        scratch_types=[
            pltpu.VMEM((S, NB_PER_W), jnp.int32),
            pltpu.VMEM((NSLOT * BCH, D), jnp.float32),
            pltpu.VMEM((NSLOT * D, TPW), jnp.float32),
            pltpu.SemaphoreType.DMA,
            pltpu.SemaphoreType.DMA((NSLOT,)),
            pltpu.SemaphoreType.DMA((NSLOT,)),
        ],
        compiler_params=pltpu.CompilerParams(
            use_tc_tiling_on_sc=False, needs_layout_passes=False
        ),
    )(_gather_kernel)
    return f(idx1d, table)


def kernel(indices, embeddings):
    idx1d = indices.astype(jnp.int32).T.reshape(-1)
    w3 = _run(idx1d, embeddings)
    return jnp.transpose(w3, (2, 0, 1))
